# baseline (speedup 1.0000x reference)
import functools

import jax
import jax.numpy as jnp
from jax import lax
from jax.experimental import pallas as pl
from jax.experimental.pallas import tpu as pltpu
from jax.experimental.pallas import tpu_sc as plsc

N = 16384
D = 64

_info = plsc.get_sparse_core_info()
NC, NS = _info.num_cores, _info.num_subcores
NW = NC * NS
B_PER_W = N // NW
CHUNK = 128
NCHUNK = B_PER_W // CHUNK
DGROUP = 4

_mesh = plsc.VectorSubcoreMesh(core_axis_name="c", subcore_axis_name="s")


@functools.partial(
    pl.kernel,
    mesh=_mesh,
    out_type=jax.ShapeDtypeStruct((D, N), jnp.float32),
    scratch_types=[
        pltpu.VMEM((B_PER_W,), jnp.int32),
        pltpu.VMEM((D, B_PER_W), jnp.float32),
        pltpu.SemaphoreType.DMA,
    ],
    compiler_params=pltpu.CompilerParams(use_tc_tiling_on_sc=False),
)
def _gather_kernel(table_hbm, idx_hbm, out_hbm, idx_v, rows_v, sem):
    wid = lax.axis_index("s") * NC + lax.axis_index("c")
    base = wid * B_PER_W
    pltpu.sync_copy(idx_hbm.at[pl.ds(base, B_PER_W)], idx_v)

    def dgroup_body(g, _):
        copies = []
        for dd in range(DGROUP):
            d = g * DGROUP + dd
            for j in range(NCHUNK):
                copies.append(
                    pltpu.async_copy(
                        table_hbm.at[d].at[idx_v.at[pl.ds(j * CHUNK, CHUNK)]],
                        rows_v.at[d, pl.ds(j * CHUNK, CHUNK)],
                        sem,
                    )
                )
        for c in copies:
            c.wait()
        return _

    lax.fori_loop(0, D // DGROUP, dgroup_body, None)
    pltpu.sync_copy(rows_v, out_hbm.at[:, pl.ds(base, B_PER_W)])


def kernel(trg, emb_table):
    out_t = _gather_kernel(emb_table.T, trg.astype(jnp.int32))
    return out_t.T[:, None, :]


# speedup vs baseline: 7.7855x; 7.7855x over previous
import functools

import jax
import jax.numpy as jnp
from jax import lax
from jax.experimental import pallas as pl
from jax.experimental.pallas import tpu as pltpu
from jax.experimental.pallas import tpu_sc as plsc

N = 16384
D = 64
VPAIR = 500000

_info = plsc.get_sparse_core_info()
NC, NS = _info.num_cores, _info.num_subcores
NW = NC * NS
B_PER_W = N // NW
CHUNK = 128
NCHUNK = B_PER_W // CHUNK
L = 16
NGROUP = B_PER_W // L

_mesh = plsc.VectorSubcoreMesh(core_axis_name="c", subcore_axis_name="s")


@functools.partial(
    pl.kernel,
    mesh=_mesh,
    out_type=jax.ShapeDtypeStruct((N, 2 * D), jnp.float32),
    scratch_types=[
        pltpu.VMEM((B_PER_W,), jnp.int32),
        pltpu.VMEM((B_PER_W, 2 * D), jnp.float32),
        pltpu.SemaphoreType.DMA,
    ],
    compiler_params=pltpu.CompilerParams(use_tc_tiling_on_sc=True),
)
def _pair_gather(table_hbm, pair_hbm, out_hbm, pair_v, g_v, sem):
    wid = lax.axis_index("s") * NC + lax.axis_index("c")
    base = wid * B_PER_W
    pltpu.sync_copy(pair_hbm.at[pl.ds(base, B_PER_W)], pair_v)
    copies = [
        pltpu.async_copy(
            table_hbm.at[pair_v.at[pl.ds(j * CHUNK, CHUNK)]],
            g_v.at[pl.ds(j * CHUNK, CHUNK)],
            sem,
        )
        for j in range(NCHUNK)
    ]
    for c in copies:
        c.wait()
    pltpu.sync_copy(g_v, out_hbm.at[pl.ds(base, B_PER_W)])


@functools.partial(
    pl.kernel,
    mesh=_mesh,
    out_type=jax.ShapeDtypeStruct((D, N), jnp.float32),
    scratch_types=[
        pltpu.VMEM((B_PER_W,), jnp.int32),
        pltpu.VMEM((B_PER_W, 2 * D), jnp.float32),
        pltpu.VMEM((D, B_PER_W), jnp.float32),
    ],
    compiler_params=pltpu.CompilerParams(needs_layout_passes=False),
)
def _half_extract(g_hbm, idx_hbm, out_hbm, idx_v, g_v, rows_v):
    wid = lax.axis_index("s") * NC + lax.axis_index("c")
    base = wid * B_PER_W
    pltpu.sync_copy(idx_hbm.at[pl.ds(base, B_PER_W)], idx_v)
    pltpu.sync_copy(g_hbm.at[pl.ds(base, B_PER_W)], g_v)
    lane_iota = lax.iota(jnp.int32, L)

    def extract_body(t, carry):
        r = idx_v[pl.ds(t * L, L)]
        half = lax.shift_left(lax.rem(r, 2), 6)
        row_ids = lane_iota + t * L
        for d in range(D):
            vals = plsc.load_gather(g_v, [row_ids, half + d])
            rows_v[d, pl.ds(t * L, L)] = vals
        return carry

    lax.fori_loop(0, NGROUP, extract_body, None)
    pltpu.sync_copy(rows_v, out_hbm.at[:, pl.ds(base, B_PER_W)])


def kernel(trg, emb_table):
    trg = trg.astype(jnp.int32)
    table2 = emb_table.reshape(VPAIR, 2 * D)
    pairs = lax.shift_right_logical(trg, 1)
    g = _pair_gather(table2, pairs)
    out_t = _half_extract(g, trg)
    return out_t.T[:, None, :]


# speedup vs baseline: 16.1154x; 2.0699x over previous
import functools

import jax
import jax.numpy as jnp
from jax import lax
from jax.experimental import pallas as pl
from jax.experimental.pallas import tpu as pltpu
from jax.experimental.pallas import tpu_sc as plsc

N = 16384
D = 64
V = 1000000
DS_W = 128
NBLK_LAST = V // 128

_info = plsc.get_sparse_core_info()
NC, NS = _info.num_cores, _info.num_subcores
NW = NC * NS
B_PER_W = N // NW
CHUNK = 128
NCHUNK = B_PER_W // CHUNK
L = 16
NGROUP = B_PER_W // L

_mesh = plsc.VectorSubcoreMesh(core_axis_name="c", subcore_axis_name="s")


@functools.partial(
    pl.kernel,
    mesh=_mesh,
    out_type=jax.ShapeDtypeStruct((N, DS_W), jnp.float32),
    scratch_types=[
        pltpu.VMEM((B_PER_W,), jnp.int32),
        pltpu.VMEM((D, 128), jnp.float32),
        pltpu.VMEM((B_PER_W, DS_W), jnp.float32),
    ],
    compiler_params=pltpu.CompilerParams(
        use_tc_tiling_on_sc=True, needs_layout_passes=False
    ),
)
def _sweep_gather(table_hbm, srt_hbm, tail_hbm, ds_hbm, rv, bbuf, rows):
    wid = lax.axis_index("s") * NC + lax.axis_index("c")
    base = wid * B_PER_W
    pltpu.sync_copy(srt_hbm.at[pl.ds(base, B_PER_W)], rv)
    lane = lax.iota(jnp.int32, L)

    def group_body(t, cur_blk):
        rvec = rv[pl.ds(t * L, L)]
        for k in range(L):
            r = lax.squeeze(lax.slice_in_dim(rvec, k, k + 1), (0,))
            c = lax.shift_right_logical(r, 7)

            @pl.when(c != cur_blk)
            def _load():
                @pl.when(c == NBLK_LAST)
                def _partial():
                    pltpu.sync_copy(tail_hbm, bbuf)

                @pl.when(c != NBLK_LAST)
                def _full():
                    off = pl.multiple_of(lax.shift_left(c, 7), 128)
                    pltpu.sync_copy(table_hbm.at[:, pl.ds(off, 128)], bbuf)

            col = jnp.full((L,), r - lax.shift_left(c, 7), jnp.int32)
            row_i = jnp.full((L,), t * L + k, jnp.int32)
            for g in range(D // L):
                vals = plsc.load_gather(bbuf, [lane + g * L, col])
                plsc.store_scatter(rows, [row_i, lane + g * L], vals)
            cur_blk = c
        return cur_blk

    lax.fori_loop(0, NGROUP, group_body, jnp.int32(-1))
    pltpu.sync_copy(rows, ds_hbm.at[pl.ds(base, B_PER_W)])


@functools.partial(
    pl.kernel,
    mesh=_mesh,
    out_type=jax.ShapeDtypeStruct((D, N), jnp.float32),
    scratch_types=[
        pltpu.VMEM((B_PER_W,), jnp.int32),
        pltpu.VMEM((B_PER_W, DS_W), jnp.float32),
        pltpu.VMEM((D, B_PER_W), jnp.float32),
        pltpu.SemaphoreType.DMA,
    ],
    compiler_params=pltpu.CompilerParams(needs_layout_passes=False),
)
def _unpermute(ds_hbm, rank_hbm, out_hbm, rank_v, gv, rt, sem):
    wid = lax.axis_index("s") * NC + lax.axis_index("c")
    base = wid * B_PER_W
    pltpu.sync_copy(rank_hbm.at[pl.ds(base, B_PER_W)], rank_v)
    copies = [
        pltpu.async_copy(
            ds_hbm.at[rank_v.at[pl.ds(j * CHUNK, CHUNK)]],
            gv.at[pl.ds(j * CHUNK, CHUNK)],
            sem,
        )
        for j in range(NCHUNK)
    ]
    for c in copies:
        c.wait()
    lane = lax.iota(jnp.int32, L)

    def extract_body(t, carry):
        row_ids = lane + t * L
        for d in range(D):
            vals = plsc.load_gather(gv, [row_ids, jnp.full((L,), d, jnp.int32)])
            rt[d, pl.ds(t * L, L)] = vals
        return carry

    lax.fori_loop(0, NGROUP, extract_body, None)
    pltpu.sync_copy(rt, out_hbm.at[:, pl.ds(base, B_PER_W)])


def kernel(trg, emb_table):
    trg_i = trg.astype(jnp.int32)
    order = jnp.argsort(trg_i).astype(jnp.int32)
    sorted_r = jnp.take(trg_i, order, axis=0)
    rank = jnp.argsort(order).astype(jnp.int32)
    tail = jnp.pad(emb_table[NBLK_LAST * 128 :].T, ((0, 0), (0, 128 - (V - NBLK_LAST * 128))))
    ds = _sweep_gather(emb_table.T, sorted_r, tail)
    out_t = _unpermute(ds, rank)
    return out_t.T[:, None, :]


# speedup vs baseline: 32.7892x; 2.0347x over previous
import functools

import jax
import jax.numpy as jnp
from jax import lax
from jax.experimental import pallas as pl
from jax.experimental.pallas import tpu as pltpu
from jax.experimental.pallas import tpu_sc as plsc

N = 16384
D = 64
V = 1000000
DS_W = 128
NBLK_LAST = V // 128

_info = plsc.get_sparse_core_info()
NC, NS = _info.num_cores, _info.num_subcores
NW = NC * NS
B_PER_W = N // NW
CHUNK = 128
NCHUNK = B_PER_W // CHUNK
L = 16
NGROUP = B_PER_W // L
PIPE = 4

_mesh = plsc.VectorSubcoreMesh(core_axis_name="c", subcore_axis_name="s")


@functools.partial(
    pl.kernel,
    mesh=_mesh,
    out_type=jax.ShapeDtypeStruct((N, DS_W), jnp.float32),
    scratch_types=[
        pltpu.VMEM((B_PER_W,), jnp.int32),
        pltpu.SMEM((B_PER_W,), jnp.int32),
        pltpu.SMEM((B_PER_W,), jnp.int32),
        pltpu.SMEM((B_PER_W + 1,), jnp.int32),
        pltpu.VMEM((D, PIPE * 128), jnp.float32),
        pltpu.VMEM((B_PER_W, DS_W), jnp.float32),
        pltpu.SemaphoreType.DMA,
    ],
    compiler_params=pltpu.CompilerParams(
        use_tc_tiling_on_sc=True, needs_layout_passes=False
    ),
)
def _sweep_gather(
    table_hbm, srt_hbm, tail_hbm, ds_hbm, rv, cols, blks, bs, bbuf, rows, sem
):
    wid = lax.axis_index("s") * NC + lax.axis_index("c")
    base = wid * B_PER_W
    pltpu.sync_copy(srt_hbm.at[pl.ds(base, B_PER_W)], rv)
    lane = lax.iota(jnp.int32, L)

    def scan_group(t, carry):
        nb, prev = carry
        rvec = rv[pl.ds(t * L, L)]
        for k in range(L):
            r = lax.squeeze(lax.slice_in_dim(rvec, k, k + 1), (0,))
            c = lax.shift_right_logical(r, 7)
            cols[t * L + k] = r - lax.shift_left(c, 7)
            is_new = c != prev

            @pl.when(is_new)
            def _record():
                blks[nb] = c
                bs[nb] = t * L + k

            nb = nb + is_new.astype(jnp.int32)
            prev = c
        return nb, prev

    nb, _ = lax.fori_loop(0, NGROUP, scan_group, (jnp.int32(0), jnp.int32(-1)))
    bs[nb] = B_PER_W

    def step(i, carry):
        @pl.when(i < nb)
        def _issue():
            c = blks[i]
            slot = pl.multiple_of((lax.rem(i, PIPE)) * 128, 128)

            @pl.when(c == NBLK_LAST)
            def _tail():
                pltpu.async_copy(tail_hbm, bbuf.at[:, pl.ds(slot, 128)], sem)

            @pl.when(c != NBLK_LAST)
            def _full():
                off = pl.multiple_of(lax.shift_left(c, 7), 128)
                pltpu.async_copy(
                    table_hbm.at[:, pl.ds(off, 128)],
                    bbuf.at[:, pl.ds(slot, 128)],
                    sem,
                )

        e = i - (PIPE - 1)

        @pl.when((e >= 0) & (e < nb))
        def _extract():
            pltpu.make_async_copy(
                table_hbm.at[:, pl.ds(0, 128)],
                bbuf.at[:, pl.ds(0, 128)],
                sem,
            ).wait()
            slot_off = lax.rem(e, PIPE) * 128

            def match_body(j, carry2):
                col = cols[j] + slot_off
                colv = jnp.full((L,), col, jnp.int32)
                row_j = jnp.full((L,), j, jnp.int32)
                for g in range(D // L):
                    vals = plsc.load_gather(bbuf, [lane + g * L, colv])
                    plsc.store_scatter(rows, [row_j, lane + g * L], vals)
                return carry2

            lax.fori_loop(bs[e], bs[e + 1], match_body, jnp.int32(0))

        return carry

    lax.fori_loop(0, nb + (PIPE - 1), step, jnp.int32(0))
    pltpu.sync_copy(rows, ds_hbm.at[pl.ds(base, B_PER_W)])


@functools.partial(
    pl.kernel,
    mesh=_mesh,
    out_type=jax.ShapeDtypeStruct((D, N), jnp.float32),
    scratch_types=[
        pltpu.VMEM((B_PER_W,), jnp.int32),
        pltpu.VMEM((B_PER_W, DS_W), jnp.float32),
        pltpu.VMEM((D, B_PER_W), jnp.float32),
        pltpu.SemaphoreType.DMA,
    ],
    compiler_params=pltpu.CompilerParams(needs_layout_passes=False),
)
def _unpermute(ds_hbm, rank_hbm, out_hbm, rank_v, gv, rt, sem):
    wid = lax.axis_index("s") * NC + lax.axis_index("c")
    base = wid * B_PER_W
    pltpu.sync_copy(rank_hbm.at[pl.ds(base, B_PER_W)], rank_v)
    copies = [
        pltpu.async_copy(
            ds_hbm.at[rank_v.at[pl.ds(j * CHUNK, CHUNK)]],
            gv.at[pl.ds(j * CHUNK, CHUNK)],
            sem,
        )
        for j in range(NCHUNK)
    ]
    for c in copies:
        c.wait()
    lane = lax.iota(jnp.int32, L)

    def extract_body(t, carry):
        row_ids = lane + t * L
        for d in range(D):
            vals = plsc.load_gather(gv, [row_ids, jnp.full((L,), d, jnp.int32)])
            rt[d, pl.ds(t * L, L)] = vals
        return carry

    lax.fori_loop(0, NGROUP, extract_body, None)
    pltpu.sync_copy(rt, out_hbm.at[:, pl.ds(base, B_PER_W)])


def kernel(trg, emb_table):
    trg_i = trg.astype(jnp.int32)
    order = jnp.argsort(trg_i).astype(jnp.int32)
    sorted_r = jnp.take(trg_i, order, axis=0)
    rank = jnp.argsort(order).astype(jnp.int32)
    tail = jnp.pad(emb_table[NBLK_LAST * 128 :].T, ((0, 0), (0, 128 - (V - NBLK_LAST * 128))))
    ds = _sweep_gather(emb_table.T, sorted_r, tail)
    out_t = _unpermute(ds, rank)
    return out_t.T[:, None, :]


# speedup vs baseline: 34.5923x; 1.0550x over previous
import functools

import jax
import jax.numpy as jnp
from jax import lax
from jax.experimental import pallas as pl
from jax.experimental.pallas import tpu as pltpu
from jax.experimental.pallas import tpu_sc as plsc

N = 16384
D = 64
V = 1000000
DS_W = 128
NBLK_LAST = V // 128

_info = plsc.get_sparse_core_info()
NC, NS = _info.num_cores, _info.num_subcores
NW = NC * NS
B_PER_W = N // NW
CHUNK = 128
NCHUNK = B_PER_W // CHUNK
L = 16
NGROUP = B_PER_W // L
PIPE = 6

_mesh = plsc.VectorSubcoreMesh(core_axis_name="c", subcore_axis_name="s")


@functools.partial(
    pl.kernel,
    mesh=_mesh,
    out_type=jax.ShapeDtypeStruct((N, DS_W), jnp.float32),
    scratch_types=[
        pltpu.VMEM((B_PER_W,), jnp.int32),
        pltpu.SMEM((B_PER_W,), jnp.int32),
        pltpu.SMEM((B_PER_W,), jnp.int32),
        pltpu.SMEM((B_PER_W + 1,), jnp.int32),
        pltpu.VMEM((D, PIPE * 128), jnp.float32),
        pltpu.VMEM((B_PER_W, DS_W), jnp.float32),
        pltpu.SemaphoreType.DMA,
    ],
    compiler_params=pltpu.CompilerParams(
        use_tc_tiling_on_sc=True, needs_layout_passes=False
    ),
)
def _sweep_gather(
    table_hbm, srt_hbm, tail_hbm, ds_hbm, rv, cols, blks, bs, bbuf, rows, sem
):
    wid = lax.axis_index("s") * NC + lax.axis_index("c")
    base = wid * B_PER_W
    pltpu.sync_copy(srt_hbm.at[pl.ds(base, B_PER_W)], rv)
    lane = lax.iota(jnp.int32, L)

    def scan_group(t, carry):
        nb, prev = carry
        rvec = rv[pl.ds(t * L, L)]
        for k in range(L):
            r = lax.squeeze(lax.slice_in_dim(rvec, k, k + 1), (0,))
            c = lax.shift_right_logical(r, 7)
            cols[t * L + k] = r - lax.shift_left(c, 7)
            is_new = c != prev

            @pl.when(is_new)
            def _record():
                blks[nb] = c
                bs[nb] = t * L + k

            nb = nb + is_new.astype(jnp.int32)
            prev = c
        return nb, prev

    nb, _ = lax.fori_loop(0, NGROUP, scan_group, (jnp.int32(0), jnp.int32(-1)))
    bs[nb] = B_PER_W

    def step(i, carry):
        @pl.when(i < nb)
        def _issue():
            c = blks[i]
            slot = pl.multiple_of((lax.rem(i, PIPE)) * 128, 128)

            @pl.when(c == NBLK_LAST)
            def _tail():
                pltpu.async_copy(tail_hbm, bbuf.at[:, pl.ds(slot, 128)], sem)

            @pl.when(c != NBLK_LAST)
            def _full():
                off = pl.multiple_of(lax.shift_left(c, 7), 128)
                pltpu.async_copy(
                    table_hbm.at[:, pl.ds(off, 128)],
                    bbuf.at[:, pl.ds(slot, 128)],
                    sem,
                )

        e = i - (PIPE - 1)

        @pl.when((e >= 0) & (e < nb))
        def _extract():
            pltpu.make_async_copy(
                table_hbm.at[:, pl.ds(0, 128)],
                bbuf.at[:, pl.ds(0, 128)],
                sem,
            ).wait()
            slot_off = lax.rem(e, PIPE) * 128

            def match_body(j, carry2):
                col = cols[j] + slot_off
                colv = jnp.full((L,), col, jnp.int32)
                row_j = jnp.full((L,), j, jnp.int32)
                for g in range(D // L):
                    vals = plsc.load_gather(bbuf, [lane + g * L, colv])
                    plsc.store_scatter(rows, [row_j, lane + g * L], vals)
                return carry2

            lax.fori_loop(bs[e], bs[e + 1], match_body, jnp.int32(0))

        return carry

    lax.fori_loop(0, nb + (PIPE - 1), step, jnp.int32(0))
    pltpu.sync_copy(rows, ds_hbm.at[pl.ds(base, B_PER_W)])


@functools.partial(
    pl.kernel,
    mesh=_mesh,
    out_type=jax.ShapeDtypeStruct((D, N), jnp.float32),
    scratch_types=[
        pltpu.VMEM((B_PER_W,), jnp.int32),
        pltpu.VMEM((B_PER_W, DS_W), jnp.float32),
        pltpu.VMEM((D, B_PER_W), jnp.float32),
        pltpu.SemaphoreType.DMA,
    ],
    compiler_params=pltpu.CompilerParams(needs_layout_passes=False),
)
def _unpermute(ds_hbm, rank_hbm, out_hbm, rank_v, gv, rt, sem):
    wid = lax.axis_index("s") * NC + lax.axis_index("c")
    base = wid * B_PER_W
    pltpu.sync_copy(rank_hbm.at[pl.ds(base, B_PER_W)], rank_v)
    copies = [
        pltpu.async_copy(
            ds_hbm.at[rank_v.at[pl.ds(j * CHUNK, CHUNK)]],
            gv.at[pl.ds(j * CHUNK, CHUNK)],
            sem,
        )
        for j in range(NCHUNK)
    ]
    for c in copies:
        c.wait()
    lane = lax.iota(jnp.int32, L)

    def extract_body(t, carry):
        row_ids = lane + t * L
        for d in range(D):
            vals = plsc.load_gather(gv, [row_ids, jnp.full((L,), d, jnp.int32)])
            rt[d, pl.ds(t * L, L)] = vals
        return carry

    lax.fori_loop(0, NGROUP, extract_body, None)
    pltpu.sync_copy(rt, out_hbm.at[:, pl.ds(base, B_PER_W)])


def kernel(trg, emb_table):
    trg_i = trg.astype(jnp.int32)
    order = jnp.argsort(trg_i).astype(jnp.int32)
    sorted_r = jnp.take(trg_i, order, axis=0)
    rank = (
        jnp.zeros((N,), jnp.int32)
        .at[order]
        .set(jnp.arange(N, dtype=jnp.int32), mode="drop", unique_indices=True)
    )
    tail = jnp.pad(emb_table[NBLK_LAST * 128 :].T, ((0, 0), (0, 128 - (V - NBLK_LAST * 128))))
    ds = _sweep_gather(emb_table.T, sorted_r, tail)
    out_t = _unpermute(ds, rank)
    return out_t.T[:, None, :]
